# baseline (speedup 1.0000x reference)
import functools

import jax
import jax.numpy as jnp
from jax import lax
from jax.experimental import pallas as pl
from jax.experimental.pallas import tpu as pltpu
from jax.experimental.pallas import tpu_sc as plsc

N = 100000
D = 64
MAX_DEPTH = 20
BLK = 128
NBLK = (N + BLK - 1) // BLK
NBUF = 2
PF = 1

TROWS = 1000
DROWS = 21

_info = plsc.get_sparse_core_info()
NC, NS = _info.num_cores, _info.num_subcores
NW = NC * NS

BASE_BLKS = NBLK // NW
EXTRA = NBLK - BASE_BLKS * NW
MAX_BLKS = BASE_BLKS + 1
WIN = MAX_BLKS * BLK

_mesh = plsc.VectorSubcoreMesh(core_axis_name="c", subcore_axis_name="s")


@functools.partial(
    pl.kernel,
    mesh=_mesh,
    out_type=jax.ShapeDtypeStruct((N * D,), jnp.float32),
    compiler_params=pltpu.CompilerParams(
        use_tc_tiling_on_sc=True, needs_layout_passes=False),
    scratch_types=[
        pltpu.VMEM((WIN,), jnp.int32),
        pltpu.VMEM((WIN,), jnp.int32),
        pltpu.VMEM((WIN,), jnp.int32),
        pltpu.VMEM((TROWS * 64,), jnp.float32),
        pltpu.VMEM((DROWS * 64,), jnp.float32),
    ]
    + [pltpu.VMEM((BLK, 128), jnp.float32)] * NBUF
    + [pltpu.VMEM((BLK * D,), jnp.float32)] * NBUF
    + [pltpu.SemaphoreType.DMA] * (1 + 2 * NBUF),
)
def _encode(x0_hbm, x1_hbm, dep_hbm, ttab, atab, dtab, out_hbm,
            xw0, xw1, dw, tv, dv,
            a0, a1, o0, o1,
            ssem, g0sem, g1sem, w0sem, w1sem):
    a_bufs = (a0, a1)
    o_bufs = (o0, o1)
    gsems = (g0sem, g1sem)
    wsems = (w0sem, w1sem)

    wid = lax.axis_index("s") * NC + lax.axis_index("c")
    first_blk = wid * BASE_BLKS + lax.min(wid, EXTRA)
    n_blk = BASE_BLKS + jnp.where(wid < EXTRA, 1, 0)
    start = lax.min(first_blk * BLK, N - WIN)

    c0 = pltpu.async_copy(x0_hbm.at[pl.ds(start, WIN)], xw0, ssem)
    c1 = pltpu.async_copy(x1_hbm.at[pl.ds(start, WIN)], xw1, ssem)
    c2 = pltpu.async_copy(dep_hbm.at[pl.ds(start, WIN)], dw, ssem)
    c3 = pltpu.async_copy(ttab, tv, ssem)
    c4 = pltpu.async_copy(dtab, dv, ssem)
    c0.wait()
    c1.wait()
    c2.wait()
    c3.wait()
    c4.wait()

    def prep(i, carry):
        s = pl.ds(i * 16, 16)
        dw[s] = jnp.minimum(dw[s], MAX_DEPTH)
        return carry

    lax.fori_loop(0, WIN // 16, prep, 0)

    def lbase(k):
        return lax.min((first_blk + k) * BLK, N - BLK) - start

    def issue(k, b):
        lb = lbase(k)
        pltpu.async_copy(atab.at[xw1.at[pl.ds(lb, BLK)]], a_bufs[b], gsems[b])

    iota = lax.iota(jnp.int32, 16)

    def compute(k, b):
        pltpu.make_async_copy(
            atab.at[xw1.at[pl.ds(0, BLK)]], a_bufs[b], gsems[b]).wait()
        lb = lbase(k)
        ab = a_bufs[b]
        ob = o_bufs[b]
        for g in range(BLK // 16):
            s = pl.ds(lb + g * 16, 16)
            tfl = jnp.left_shift(xw0[s], 6)
            dfl = jnp.left_shift(dw[s], 6)

            def row_body(lane, carry, g=g, tfl=tfl, dfl=dfl):
                lv = jnp.full((16,), lane, jnp.int32)
                ts = tfl.at[lv].get(mode="promise_in_bounds")
                dsp = dfl.at[lv].get(mode="promise_in_bounds")
                row = g * 16 + lane
                for c4 in range(D // 16):
                    co = c4 * 16
                    av = ab[row, pl.ds(co, 16)]
                    tvv = plsc.load_gather(tv, [ts + co + iota])
                    dvv = plsc.load_gather(dv, [dsp + co + iota])
                    ob[pl.ds(row * D + co, 16)] = av + tvv + dvv
                return carry

            lax.fori_loop(0, 16, row_body, 0)
        gb = lax.min((first_blk + k) * BLK, N - BLK)
        pltpu.async_copy(ob, out_hbm.at[pl.ds(gb * D, BLK * D)], wsems[b])

    for k in range(PF):
        issue(k, k)

    for i in range(NBUF):
        @pl.when(i + PF < n_blk)
        def _(i=i):
            issue(i + PF, (i + PF) % NBUF)

        @pl.when(i < n_blk)
        def _(i=i):
            compute(i, i)

    n_grp = (MAX_BLKS + NBUF - 1) // NBUF

    def group(gidx, carry):
        for i in range(NBUF):
            k = gidx * NBUF + i

            @pl.when(k + PF < n_blk)
            def _(i=i, k=k):
                issue(k + PF, (i + PF) % NBUF)

            @pl.when(k < n_blk)
            def _(i=i, k=k):
                pltpu.make_async_copy(
                    o_bufs[i], out_hbm.at[pl.ds(0, BLK * D)], wsems[i]).wait()
                compute(k, i)

        return carry

    lax.fori_loop(1, n_grp, group, 0)

    for kk in range(BASE_BLKS - NBUF, MAX_BLKS):
        b = kk % NBUF

        def drain(b=b):
            pltpu.make_async_copy(
                o_bufs[b], out_hbm.at[pl.ds(0, BLK * D)], wsems[b]).wait()

        pl.when(jnp.logical_and(kk >= n_blk - NBUF, kk < n_blk))(drain)


def kernel(x, depth, type_table, attribute_table, depth_table):
    x0 = x[:, 0]
    x1 = x[:, 1]
    t2 = type_table.reshape(TROWS * 64)
    a2 = jnp.pad(attribute_table.T, ((0, 64), (0, 0))).T
    d2 = depth_table.reshape(DROWS * 64)
    out = _encode(x0, x1, depth, t2, a2, d2)
    return out.reshape(N, D)


# speedup vs baseline: 1.0009x; 1.0009x over previous
import functools

import jax
import jax.numpy as jnp
from jax import lax
from jax.experimental import pallas as pl
from jax.experimental.pallas import tpu as pltpu
from jax.experimental.pallas import tpu_sc as plsc

N = 100000
D = 64
MAX_DEPTH = 20
BLK = 128
NBLK = (N + BLK - 1) // BLK
NBUF = 2
PF = 1

TROWS = 1000
DROWS = 21

_info = plsc.get_sparse_core_info()
NC, NS = _info.num_cores, _info.num_subcores
NW = NC * NS

BASE_BLKS = NBLK // NW
EXTRA = NBLK - BASE_BLKS * NW
MAX_BLKS = BASE_BLKS + 1
WIN = MAX_BLKS * BLK

_mesh = plsc.VectorSubcoreMesh(core_axis_name="c", subcore_axis_name="s")


@functools.partial(
    pl.kernel,
    mesh=_mesh,
    out_type=jax.ShapeDtypeStruct((N * D,), jnp.float32),
    compiler_params=pltpu.CompilerParams(
        use_tc_tiling_on_sc=True, needs_layout_passes=False),
    scratch_types=[
        pltpu.VMEM((WIN,), jnp.int32),
        pltpu.VMEM((WIN,), jnp.int32),
        pltpu.VMEM((WIN,), jnp.int32),
        pltpu.VMEM((TROWS * 64,), jnp.float32),
        pltpu.VMEM((DROWS * 64,), jnp.float32),
    ]
    + [pltpu.VMEM((BLK, 128), jnp.float32)] * NBUF
    + [pltpu.VMEM((BLK * D,), jnp.float32)] * NBUF
    + [pltpu.SemaphoreType.DMA] * (1 + 2 * NBUF),
)
def _encode(x0_hbm, x1_hbm, dep_hbm, ttab, atab, dtab, out_hbm,
            xw0, xw1, dw, tv, dv,
            a0, a1, o0, o1,
            ssem, g0sem, g1sem, w0sem, w1sem):
    a_bufs = (a0, a1)
    o_bufs = (o0, o1)
    gsems = (g0sem, g1sem)
    wsems = (w0sem, w1sem)

    wid = lax.axis_index("s") * NC + lax.axis_index("c")
    first_blk = wid * BASE_BLKS + lax.min(wid, EXTRA)
    n_blk = BASE_BLKS + jnp.where(wid < EXTRA, 1, 0)
    start = lax.min(first_blk * BLK, N - WIN)

    c0 = pltpu.async_copy(x0_hbm.at[pl.ds(start, WIN)], xw0, ssem)
    c1 = pltpu.async_copy(x1_hbm.at[pl.ds(start, WIN)], xw1, ssem)
    c2 = pltpu.async_copy(dep_hbm.at[pl.ds(start, WIN)], dw, ssem)
    c3 = pltpu.async_copy(ttab, tv, ssem)
    c4 = pltpu.async_copy(dtab, dv, ssem)
    c0.wait()
    c1.wait()
    c2.wait()
    c3.wait()
    c4.wait()

    def prep(i, carry):
        s = pl.ds(i * 16, 16)
        dw[s] = jnp.minimum(dw[s], MAX_DEPTH)
        return carry

    lax.fori_loop(0, WIN // 16, prep, 0)

    def lbase(k):
        return lax.min((first_blk + k) * BLK, N - BLK) - start

    def issue(k, b):
        lb = lbase(k)
        pltpu.async_copy(atab.at[xw1.at[pl.ds(lb, BLK)]], a_bufs[b], gsems[b])

    iota = lax.iota(jnp.int32, 16)

    def compute(k, b):
        pltpu.make_async_copy(
            atab.at[xw1.at[pl.ds(0, BLK)]], a_bufs[b], gsems[b]).wait()
        lb = lbase(k)
        ab = a_bufs[b]
        ob = o_bufs[b]
        for g in range(BLK // 16):
            s = pl.ds(lb + g * 16, 16)
            tfl = jnp.left_shift(xw0[s], 6)
            dfl = jnp.left_shift(dw[s], 6)

            def row_body(lane, carry, g=g, tfl=tfl, dfl=dfl):
                lv = jnp.full((16,), lane, jnp.int32)
                ts = tfl.at[lv].get(mode="promise_in_bounds")
                dsp = dfl.at[lv].get(mode="promise_in_bounds")
                row = g * 16 + lane
                for c4 in range(D // 16):
                    co = c4 * 16
                    av = ab[row, pl.ds(co, 16)]
                    tvv = plsc.load_gather(tv, [ts + co + iota])
                    dvv = plsc.load_gather(dv, [dsp + co + iota])
                    ob[pl.ds(row * D + co, 16)] = av + tvv + dvv
                return carry

            lax.fori_loop(0, 16, row_body, 0)
        gb = lax.min((first_blk + k) * BLK, N - BLK)
        pltpu.async_copy(ob, out_hbm.at[pl.ds(gb * D, BLK * D)], wsems[b])

    for k in range(PF):
        issue(k, k)

    for i in range(NBUF):
        @pl.when(i + PF < n_blk)
        def _(i=i):
            issue(i + PF, (i + PF) % NBUF)

        @pl.when(i < n_blk)
        def _(i=i):
            compute(i, i)

    n_grp = (MAX_BLKS + NBUF - 1) // NBUF

    def group(gidx, carry):
        for i in range(NBUF):
            k = gidx * NBUF + i

            @pl.when(k + PF < n_blk)
            def _(i=i, k=k):
                issue(k + PF, (i + PF) % NBUF)

            @pl.when(k < n_blk)
            def _(i=i, k=k):
                pltpu.make_async_copy(
                    o_bufs[i], out_hbm.at[pl.ds(0, BLK * D)], wsems[i]).wait()
                compute(k, i)

        return carry

    lax.fori_loop(1, n_grp, group, 0)

    for kk in range(BASE_BLKS - NBUF, MAX_BLKS):
        b = kk % NBUF

        def drain(b=b):
            pltpu.make_async_copy(
                o_bufs[b], out_hbm.at[pl.ds(0, BLK * D)], wsems[b]).wait()

        pl.when(jnp.logical_and(kk >= n_blk - NBUF, kk < n_blk))(drain)


def kernel(x, depth, type_table, attribute_table, depth_table):
    x0 = x[:, 0]
    x1 = x[:, 1]
    t2 = type_table.reshape(TROWS * 64)
    a2 = jnp.pad(attribute_table, ((0, 0), (0, 64)))
    d2 = depth_table.reshape(DROWS * 64)
    out = _encode(x0, x1, depth, t2, a2, d2)
    return out.reshape(N, D)
